# Initial kernel scaffold; baseline (speedup 1.0000x reference)
import jax
import jax.numpy as jnp
from jax.experimental import pallas as pl


def kernel(X, w0, bias_table, emb_table):
    raise NotImplementedError("write your pallas kernel here")


# speedup vs baseline: 2.1409x; 2.1409x over previous
import functools

import jax
import jax.numpy as jnp
from jax import lax
from jax.experimental import pallas as pl
from jax.experimental.pallas import tpu as pltpu
from jax.experimental.pallas import tpu_sc as plsc

N_VOCAB = 1000000
EMBED_DIM = 32
BATCH = 16384
N_FIELDS = 26

NC = 2
NS = 16
NW = NC * NS
L = 16

S_PER_W = BATCH // NW
CHUNK = 32
N_CHUNKS = S_PER_W // CHUNK
IDX_PER_CHUNK = CHUNK * N_FIELDS
IDX_PER_W = S_PER_W * N_FIELDS
SUB_FULL = IDX_PER_CHUNK // 128
SUB_REM = IDX_PER_CHUNK - SUB_FULL * 128


def _fm_body(x_hbm, w0_hbm, bias_hbm, emb_hbm, out_hbm,
             xidx, embv, biasv, logits, w0v, sem):
    wid = lax.axis_index("s") * NC + lax.axis_index("c")
    base_idx = pl.multiple_of(wid * IDX_PER_W, IDX_PER_W)

    pltpu.sync_copy(x_hbm.at[pl.ds(base_idx, IDX_PER_W)], xidx)
    pltpu.sync_copy(w0_hbm, w0v)

    lane = jnp.arange(L, dtype=jnp.int32)
    lane0 = lane == 0
    tail_mask = jnp.where(lane < (N_FIELDS - L), 1.0, 0.0).astype(jnp.float32)
    tail_idx_off = jnp.minimum(lane, N_FIELDS - L - 1) + L

    @pl.loop(0, N_CHUNKS)
    def _chunk(c):
        coff = pl.multiple_of(c * IDX_PER_CHUNK, IDX_PER_CHUNK)
        copies = []
        for j in range(SUB_FULL + 1):
            n = 128 if j < SUB_FULL else SUB_REM
            o = j * 128
            idx_ref = xidx.at[pl.ds(coff + o, n)]
            copies.append(pltpu.async_copy(
                emb_hbm.at[idx_ref], embv.at[pl.ds(o, n)], sem))
            copies.append(pltpu.async_copy(
                bias_hbm.at[idx_ref], biasv.at[pl.ds(o, n)], sem))
        for cp in copies:
            cp.wait()

        @pl.loop(0, CHUNK)
        def _sample(i):
            kb = i * N_FIELDS
            s0 = jnp.zeros((L,), jnp.float32)
            s1 = jnp.zeros((L,), jnp.float32)
            q0 = jnp.zeros((L,), jnp.float32)
            q1 = jnp.zeros((L,), jnp.float32)
            for f in range(N_FIELDS):
                r0 = embv[kb + f, pl.ds(0, L)]
                r1 = embv[kb + f, pl.ds(L, L)]
                s0 = s0 + r0
                q0 = q0 + r0 * r0
                s1 = s1 + r1
                q1 = q1 + r1 * r1
            u = (s0 * s0 - q0) + (s1 * s1 - q1)
            b0 = plsc.load_gather(biasv, [kb + lane])
            b1 = plsc.load_gather(biasv, [kb + tail_idx_off]) * tail_mask
            acc = 0.5 * u + b0 + b1
            r = jnp.sum(acc)
            g = c * CHUNK + i
            plsc.store_scatter(
                logits, [jnp.broadcast_to(g, (L,)).astype(jnp.int32)],
                jnp.broadcast_to(r, (L,)).astype(jnp.float32), mask=lane0)

    w0vec = w0v[...]

    @pl.loop(0, S_PER_W // L)
    def _sig(g):
        off = pl.multiple_of(g * L, L)
        z = logits[pl.ds(off, L)] + w0vec
        logits[pl.ds(off, L)] = 5.5 / (1.0 + jnp.exp(-z))

    out_base = pl.multiple_of(wid * S_PER_W, S_PER_W)
    pltpu.sync_copy(logits, out_hbm.at[pl.ds(out_base, S_PER_W)])


_fm_call = pl.kernel(
    _fm_body,
    out_type=jax.ShapeDtypeStruct((BATCH,), jnp.float32),
    mesh=plsc.VectorSubcoreMesh(core_axis_name="c", subcore_axis_name="s"),
    scratch_types=[
        pltpu.VMEM((IDX_PER_W,), jnp.int32),
        pltpu.VMEM((IDX_PER_CHUNK, EMBED_DIM), jnp.float32),
        pltpu.VMEM((IDX_PER_CHUNK,), jnp.float32),
        pltpu.VMEM((S_PER_W,), jnp.float32),
        pltpu.VMEM((L,), jnp.float32),
        pltpu.SemaphoreType.DMA,
    ],
    compiler_params=pltpu.CompilerParams(
        needs_layout_passes=False, use_tc_tiling_on_sc=False),
)


def kernel(X, w0, bias_table, emb_table):
    x_flat = X.reshape(-1).astype(jnp.int32)
    bias_flat = bias_table.reshape(-1)
    w0v = jnp.broadcast_to(w0.astype(jnp.float32), (L,))
    return _fm_call(x_flat, w0v, bias_flat, emb_table)


# speedup vs baseline: 2.2304x; 1.0418x over previous
import functools

import jax
import jax.numpy as jnp
from jax import lax
from jax.experimental import pallas as pl
from jax.experimental.pallas import tpu as pltpu
from jax.experimental.pallas import tpu_sc as plsc

N_VOCAB = 1000000
EMBED_DIM = 32
BATCH = 16384
N_FIELDS = 26

NC = 2
NS = 16
NW = NC * NS
L = 16

S_PER_W = BATCH // NW
CHUNK = 32
N_CHUNKS = S_PER_W // CHUNK
IDX_PER_CHUNK = CHUNK * N_FIELDS
IDX_PER_W = S_PER_W * N_FIELDS
SUB_FULL = IDX_PER_CHUNK // 128
SUB_REM = IDX_PER_CHUNK - SUB_FULL * 128


def _fm_body(x_hbm, w0_hbm, bias_hbm, emb_hbm, out_hbm,
             xidx, embv, biasv, logits, w0v, sem):
    wid = lax.axis_index("s") * NC + lax.axis_index("c")
    base_idx = pl.multiple_of(wid * IDX_PER_W, IDX_PER_W)

    pltpu.sync_copy(x_hbm.at[pl.ds(base_idx, IDX_PER_W)], xidx)
    pltpu.sync_copy(w0_hbm, w0v)

    lane = jnp.arange(L, dtype=jnp.int32)
    lane0 = lane == 0
    tail_mask = jnp.where(lane < (N_FIELDS - L), 1.0, 0.0).astype(jnp.float32)
    tail_idx_off = jnp.minimum(lane, N_FIELDS - L - 1) + L

    def _streams(c, b):
        coff = pl.multiple_of(c * IDX_PER_CHUNK, IDX_PER_CHUNK)
        ops = []
        for j in range(SUB_FULL + 1):
            n = 128 if j < SUB_FULL else SUB_REM
            o = j * 128
            idx_ref = xidx.at[pl.ds(coff + o, n)]
            ops.append((emb_hbm.at[idx_ref], embv.at[b, pl.ds(o, n)]))
            ops.append((bias_hbm.at[idx_ref], biasv.at[b, pl.ds(o, n)]))
        return ops

    def _issue(c, b):
        for src, dst in _streams(c, b):
            pltpu.async_copy(src, dst, sem)

    def _drain(c, b):
        for src, dst in _streams(c, b):
            pltpu.make_async_copy(src, dst, sem).wait()

    _issue(0, 0)

    @pl.loop(0, N_CHUNKS)
    def _chunk(c):
        b = lax.rem(c, 2)
        _drain(c, b)

        @pl.when(c + 1 < N_CHUNKS)
        def _():
            _issue(c + 1, 1 - b)

        @pl.loop(0, CHUNK)
        def _sample(i):
            kb = i * N_FIELDS
            s0 = jnp.zeros((L,), jnp.float32)
            s1 = jnp.zeros((L,), jnp.float32)
            q0 = jnp.zeros((L,), jnp.float32)
            q1 = jnp.zeros((L,), jnp.float32)
            for f in range(N_FIELDS):
                r0 = embv[b, kb + f, pl.ds(0, L)]
                r1 = embv[b, kb + f, pl.ds(L, L)]
                s0 = s0 + r0
                q0 = q0 + r0 * r0
                s1 = s1 + r1
                q1 = q1 + r1 * r1
            u = (s0 * s0 - q0) + (s1 * s1 - q1)
            b0 = plsc.load_gather(biasv.at[b], [kb + lane])
            b1 = plsc.load_gather(biasv.at[b], [kb + tail_idx_off]) * tail_mask
            acc = 0.5 * u + b0 + b1
            r = jnp.sum(acc)
            g = c * CHUNK + i
            plsc.store_scatter(
                logits, [jnp.broadcast_to(g, (L,)).astype(jnp.int32)],
                jnp.broadcast_to(r, (L,)).astype(jnp.float32), mask=lane0)

    w0vec = w0v[...]

    @pl.loop(0, S_PER_W // L)
    def _sig(g):
        off = pl.multiple_of(g * L, L)
        z = logits[pl.ds(off, L)] + w0vec
        logits[pl.ds(off, L)] = 5.5 / (1.0 + jnp.exp(-z))

    out_base = pl.multiple_of(wid * S_PER_W, S_PER_W)
    pltpu.sync_copy(logits, out_hbm.at[pl.ds(out_base, S_PER_W)])


_fm_call = pl.kernel(
    _fm_body,
    out_type=jax.ShapeDtypeStruct((BATCH,), jnp.float32),
    mesh=plsc.VectorSubcoreMesh(core_axis_name="c", subcore_axis_name="s"),
    scratch_types=[
        pltpu.VMEM((IDX_PER_W,), jnp.int32),
        pltpu.VMEM((2, IDX_PER_CHUNK, EMBED_DIM), jnp.float32),
        pltpu.VMEM((2, IDX_PER_CHUNK), jnp.float32),
        pltpu.VMEM((S_PER_W,), jnp.float32),
        pltpu.VMEM((L,), jnp.float32),
        pltpu.SemaphoreType.DMA,
    ],
    compiler_params=pltpu.CompilerParams(
        needs_layout_passes=False, use_tc_tiling_on_sc=False),
)


def kernel(X, w0, bias_table, emb_table):
    x_flat = X.reshape(-1).astype(jnp.int32)
    bias_flat = bias_table.reshape(-1)
    w0v = jnp.broadcast_to(w0.astype(jnp.float32), (L,))
    return _fm_call(x_flat, w0v, bias_flat, emb_table)


# speedup vs baseline: 2.2334x; 1.0013x over previous
import functools

import jax
import jax.numpy as jnp
from jax import lax
from jax.experimental import pallas as pl
from jax.experimental.pallas import tpu as pltpu
from jax.experimental.pallas import tpu_sc as plsc

N_VOCAB = 1000000
EMBED_DIM = 32
BATCH = 16384
N_FIELDS = 26

NC = 2
NS = 16
NW = NC * NS
L = 16

S_PER_W = BATCH // NW
CHUNK = 32
N_CHUNKS = S_PER_W // CHUNK
IDX_PER_CHUNK = CHUNK * N_FIELDS
IDX_PER_W = S_PER_W * N_FIELDS
SUB_FULL = IDX_PER_CHUNK // 128
SUB_REM = IDX_PER_CHUNK - SUB_FULL * 128


def _fm_body(x_hbm, w0_hbm, bias_hbm, emb_hbm, out_hbm,
             xidx, embv, biasv, logits, stage, w0v, sem):
    wid = lax.axis_index("s") * NC + lax.axis_index("c")
    base_idx = pl.multiple_of(wid * IDX_PER_W, IDX_PER_W)

    pltpu.sync_copy(x_hbm.at[pl.ds(base_idx, IDX_PER_W)], xidx)
    pltpu.sync_copy(w0_hbm, w0v)

    lane = jnp.arange(L, dtype=jnp.int32)

    def _streams(c, b):
        coff = pl.multiple_of(c * IDX_PER_CHUNK, IDX_PER_CHUNK)
        ops = []
        for j in range(SUB_FULL + 1):
            n = 128 if j < SUB_FULL else SUB_REM
            o = j * 128
            idx_ref = xidx.at[pl.ds(coff + o, n)]
            ops.append((emb_hbm.at[idx_ref], embv.at[b, pl.ds(o, n)]))
            ops.append((bias_hbm.at[idx_ref], biasv.at[b, pl.ds(o, n)]))
        return ops

    def _issue(c, b):
        for src, dst in _streams(c, b):
            pltpu.async_copy(src, dst, sem)

    def _drain(c, b):
        for src, dst in _streams(c, b):
            pltpu.make_async_copy(src, dst, sem).wait()

    _issue(0, 0)
    w0vec = w0v[...]

    @pl.loop(0, N_CHUNKS)
    def _chunk(c):
        b = lax.rem(c, 2)
        _drain(c, b)

        @pl.when(c + 1 < N_CHUNKS)
        def _():
            _issue(c + 1, 1 - b)

        @pl.loop(0, CHUNK)
        def _sample(i):
            kb = i * N_FIELDS
            s0 = jnp.zeros((L,), jnp.float32)
            s1 = jnp.zeros((L,), jnp.float32)
            q0 = jnp.zeros((L,), jnp.float32)
            q1 = jnp.zeros((L,), jnp.float32)
            for f in range(N_FIELDS):
                r0 = embv[b, kb + f, pl.ds(0, L)]
                r1 = embv[b, kb + f, pl.ds(L, L)]
                s0 = s0 + r0
                q0 = q0 + r0 * r0
                s1 = s1 + r1
                q1 = q1 + r1 * r1
            u = (s0 * s0 - q0) + (s1 * s1 - q1)
            stage[i, pl.ds(0, L)] = u

        for g in range(CHUNK // L):
            rows = g * L + lane
            pair = jnp.zeros((L,), jnp.float32)
            for d in range(L):
                pair = pair + plsc.load_gather(
                    stage, [rows, jnp.full((L,), d, jnp.int32)])
            bsum = jnp.zeros((L,), jnp.float32)
            brow = rows * N_FIELDS
            for f in range(N_FIELDS):
                bsum = bsum + plsc.load_gather(biasv.at[b], [brow + f])
            z = 0.5 * pair + bsum + w0vec
            out16 = 5.5 / (1.0 + jnp.exp(-z))
            off = pl.multiple_of(c * CHUNK + g * L, L)
            logits[pl.ds(off, L)] = out16

    out_base = pl.multiple_of(wid * S_PER_W, S_PER_W)
    pltpu.sync_copy(logits, out_hbm.at[pl.ds(out_base, S_PER_W)])


_fm_call = pl.kernel(
    _fm_body,
    out_type=jax.ShapeDtypeStruct((BATCH,), jnp.float32),
    mesh=plsc.VectorSubcoreMesh(core_axis_name="c", subcore_axis_name="s"),
    scratch_types=[
        pltpu.VMEM((IDX_PER_W,), jnp.int32),
        pltpu.VMEM((2, IDX_PER_CHUNK, EMBED_DIM), jnp.float32),
        pltpu.VMEM((2, IDX_PER_CHUNK), jnp.float32),
        pltpu.VMEM((S_PER_W,), jnp.float32),
        pltpu.VMEM((CHUNK, 17), jnp.float32),
        pltpu.VMEM((L,), jnp.float32),
        pltpu.SemaphoreType.DMA,
    ],
    compiler_params=pltpu.CompilerParams(
        needs_layout_passes=False, use_tc_tiling_on_sc=False),
)


def kernel(X, w0, bias_table, emb_table):
    x_flat = X.reshape(-1).astype(jnp.int32)
    bias_flat = bias_table.reshape(-1)
    w0v = jnp.broadcast_to(w0.astype(jnp.float32), (L,))
    return _fm_call(x_flat, w0v, bias_flat, emb_table)


# speedup vs baseline: 2.2431x; 1.0043x over previous
import functools

import jax
import jax.numpy as jnp
from jax import lax
from jax.experimental import pallas as pl
from jax.experimental.pallas import tpu as pltpu
from jax.experimental.pallas import tpu_sc as plsc

N_VOCAB = 1000000
EMBED_DIM = 32
BATCH = 16384
N_FIELDS = 26

NC = 2
NS = 16
NW = NC * NS
L = 16

S_PER_W = BATCH // NW
CHUNK = 32
N_CHUNKS = S_PER_W // CHUNK
IDX_PER_CHUNK = CHUNK * N_FIELDS
IDX_PER_W = S_PER_W * N_FIELDS
SUB_N = IDX_PER_CHUNK
SUBS = IDX_PER_CHUNK // SUB_N


def _fm_body(x_hbm, w0_hbm, bias_hbm, emb_hbm, out_hbm,
             xidx, embv, biasv, logits, stage, w0v, sem):
    wid = lax.axis_index("s") * NC + lax.axis_index("c")
    base_idx = pl.multiple_of(wid * IDX_PER_W, IDX_PER_W)

    pltpu.sync_copy(x_hbm.at[pl.ds(base_idx, IDX_PER_W)], xidx)
    pltpu.sync_copy(w0_hbm, w0v)

    lane = jnp.arange(L, dtype=jnp.int32)

    def _streams(c, b):
        coff = pl.multiple_of(c * IDX_PER_CHUNK, IDX_PER_CHUNK)
        ops = []
        for j in range(SUBS):
            n = SUB_N
            o = j * SUB_N
            idx_ref = xidx.at[pl.ds(coff + o, n)]
            ops.append((emb_hbm.at[idx_ref], embv.at[b, pl.ds(o, n)]))
            ops.append((bias_hbm.at[idx_ref], biasv.at[b, pl.ds(o, n)]))
        return ops

    def _issue(c, b):
        for src, dst in _streams(c, b):
            pltpu.async_copy(src, dst, sem)

    def _drain(c, b):
        for src, dst in _streams(c, b):
            pltpu.make_async_copy(src, dst, sem).wait()

    _issue(0, 0)
    w0vec = w0v[...]

    @pl.loop(0, N_CHUNKS)
    def _chunk(c):
        b = lax.rem(c, 2)
        _drain(c, b)

        @pl.when(c + 1 < N_CHUNKS)
        def _():
            _issue(c + 1, 1 - b)

        @pl.loop(0, CHUNK)
        def _sample(i):
            kb = i * N_FIELDS
            s0 = jnp.zeros((L,), jnp.float32)
            s1 = jnp.zeros((L,), jnp.float32)
            q0 = jnp.zeros((L,), jnp.float32)
            q1 = jnp.zeros((L,), jnp.float32)
            for f in range(N_FIELDS):
                r0 = embv[b, kb + f, pl.ds(0, L)]
                r1 = embv[b, kb + f, pl.ds(L, L)]
                s0 = s0 + r0
                q0 = q0 + r0 * r0
                s1 = s1 + r1
                q1 = q1 + r1 * r1
            u = (s0 * s0 - q0) + (s1 * s1 - q1)
            stage[i, pl.ds(0, L)] = u

        for g in range(CHUNK // L):
            rows = g * L + lane
            pair = jnp.zeros((L,), jnp.float32)
            for d in range(L):
                pair = pair + plsc.load_gather(
                    stage, [rows, jnp.full((L,), d, jnp.int32)])
            bsum = jnp.zeros((L,), jnp.float32)
            brow = rows * N_FIELDS
            for f in range(N_FIELDS):
                bsum = bsum + plsc.load_gather(biasv.at[b], [brow + f])
            z = 0.5 * pair + bsum + w0vec
            out16 = 5.5 / (1.0 + jnp.exp(-z))
            off = pl.multiple_of(c * CHUNK + g * L, L)
            logits[pl.ds(off, L)] = out16

    out_base = pl.multiple_of(wid * S_PER_W, S_PER_W)
    pltpu.sync_copy(logits, out_hbm.at[pl.ds(out_base, S_PER_W)])


_fm_call = pl.kernel(
    _fm_body,
    out_type=jax.ShapeDtypeStruct((BATCH,), jnp.float32),
    mesh=plsc.VectorSubcoreMesh(core_axis_name="c", subcore_axis_name="s"),
    scratch_types=[
        pltpu.VMEM((IDX_PER_W,), jnp.int32),
        pltpu.VMEM((2, IDX_PER_CHUNK, EMBED_DIM), jnp.float32),
        pltpu.VMEM((2, IDX_PER_CHUNK), jnp.float32),
        pltpu.VMEM((S_PER_W,), jnp.float32),
        pltpu.VMEM((CHUNK, 17), jnp.float32),
        pltpu.VMEM((L,), jnp.float32),
        pltpu.SemaphoreType.DMA,
    ],
    compiler_params=pltpu.CompilerParams(
        needs_layout_passes=False, use_tc_tiling_on_sc=False),
)


def kernel(X, w0, bias_table, emb_table):
    x_flat = X.reshape(-1).astype(jnp.int32)
    bias_flat = bias_table.reshape(-1)
    w0v = jnp.broadcast_to(w0.astype(jnp.float32), (L,))
    return _fm_call(x_flat, w0v, bias_flat, emb_table)
